# baseline (device time: 385322 ns/iter reference)
import jax
import jax.numpy as jnp
from jax import lax
from jax.experimental import pallas as pl
from jax.experimental.pallas import tpu as pltpu

N_DEV = 16
SQ = 2048
SKV = 2048
D_MODEL = 1024
H_LOC = 8
DH = 128
BQ = 256
N_QBLK = SQ // BQ
CHUNK = SQ // N_DEV
SCALE = 0.08838834764831843


def _attn_body(x_ref, wq_ref, k_ref, v_ref, wo_ref, out_ref):
    q = jnp.dot(x_ref[...], wq_ref[...], preferred_element_type=jnp.float32)

    rows = pl.program_id(0) * BQ + lax.broadcasted_iota(jnp.int32, (BQ, SKV), 0)
    cols = lax.broadcasted_iota(jnp.int32, (BQ, SKV), 1)
    qb = rows // 64
    kb = cols // 64
    mask = (qb == kb) | (kb == 0) | ((qb + kb) % 3 == 0)

    acc = jnp.zeros((BQ, D_MODEL), jnp.float32)
    for h in range(H_LOC):
        qh = q[:, h * DH:(h + 1) * DH]
        kh = k_ref[h]
        s = lax.dot_general(
            qh, kh, (((1,), (1,)), ((), ())),
            preferred_element_type=jnp.float32,
        ) * SCALE
        s = jnp.where(mask, s, -1e9)
        m = jnp.max(s, axis=1, keepdims=True)
        w = jnp.exp(s - m)
        w = w / jnp.sum(w, axis=1, keepdims=True)
        ctx = jnp.dot(w, v_ref[h], preferred_element_type=jnp.float32)
        acc = acc + jnp.dot(ctx, wo_ref[h], preferred_element_type=jnp.float32)
    out_ref[...] = acc


def _allreduce_body(p_ref, out_ref, recv_ref, rs_sems, ag_sems, send_sem):
    d = lax.axis_index("i")
    left = (d - 1) % N_DEV
    right = (d + 1) % N_DEV

    barrier_sem = pltpu.get_barrier_semaphore()
    for nbr in [left, right]:
        pl.semaphore_signal(
            barrier_sem, inc=1,
            device_id=(nbr,), device_id_type=pl.DeviceIdType.MESH,
        )
    pl.semaphore_wait(barrier_sem, 2)

    out_ref[...] = p_ref[...]

    for h in range(N_DEV - 1):
        s = (d - h) % N_DEV
        rdma = pltpu.make_async_remote_copy(
            src_ref=out_ref.at[s],
            dst_ref=recv_ref.at[h],
            send_sem=send_sem,
            recv_sem=rs_sems.at[h],
            device_id=(right,),
            device_id_type=pl.DeviceIdType.MESH,
        )
        rdma.start()
        rdma.wait()
        sl = (d - 1 - h) % N_DEV
        out_ref[sl] = out_ref[sl] + recv_ref[h]

    for h in range(N_DEV - 1):
        s = (d + 1 - h) % N_DEV
        rdma = pltpu.make_async_remote_copy(
            src_ref=out_ref.at[s],
            dst_ref=out_ref.at[s],
            send_sem=send_sem,
            recv_sem=ag_sems.at[h],
            device_id=(right,),
            device_id_type=pl.DeviceIdType.MESH,
        )
        rdma.start()
        rdma.wait()


def kernel(x, Wq, K_ext, V_ext, Wo):
    d = lax.axis_index("i")

    x2 = x.reshape(SQ, D_MODEL)
    wq_loc = lax.dynamic_slice(Wq, (0, d * H_LOC * DH), (D_MODEL, H_LOC * DH))
    wo_loc = lax.dynamic_slice(
        Wo, (d * H_LOC * DH, 0), (H_LOC * DH, D_MODEL)
    ).reshape(H_LOC, DH, D_MODEL)
    k_loc = K_ext.reshape(SKV, H_LOC, DH).transpose(1, 0, 2)
    v_loc = V_ext.reshape(SKV, H_LOC, DH).transpose(1, 0, 2)

    partial = pl.pallas_call(
        _attn_body,
        grid=(N_QBLK,),
        in_specs=[
            pl.BlockSpec((BQ, D_MODEL), lambda i: (i, 0)),
            pl.BlockSpec((D_MODEL, H_LOC * DH), lambda i: (0, 0)),
            pl.BlockSpec((H_LOC, SKV, DH), lambda i: (0, 0, 0)),
            pl.BlockSpec((H_LOC, SKV, DH), lambda i: (0, 0, 0)),
            pl.BlockSpec((H_LOC, DH, D_MODEL), lambda i: (0, 0, 0)),
        ],
        out_specs=pl.BlockSpec((BQ, D_MODEL), lambda i: (i, 0)),
        out_shape=jax.ShapeDtypeStruct((SQ, D_MODEL), jnp.float32),
    )(x2, wq_loc, k_loc, v_loc, wo_loc)

    reduced = pl.pallas_call(
        _allreduce_body,
        in_specs=[pl.BlockSpec(memory_space=pltpu.VMEM)],
        out_specs=pl.BlockSpec(memory_space=pltpu.VMEM),
        out_shape=jax.ShapeDtypeStruct((N_DEV, CHUNK, D_MODEL), jnp.float32),
        scratch_shapes=[
            pltpu.VMEM((N_DEV - 1, CHUNK, D_MODEL), jnp.float32),
            pltpu.SemaphoreType.DMA((N_DEV - 1,)),
            pltpu.SemaphoreType.DMA((N_DEV - 1,)),
            pltpu.SemaphoreType.DMA,
        ],
        compiler_params=pltpu.CompilerParams(collective_id=0),
    )(partial.reshape(N_DEV, CHUNK, D_MODEL))

    return reduced.reshape(1, SQ, D_MODEL)
